# Initial kernel scaffold; baseline (speedup 1.0000x reference)
import jax
import jax.numpy as jnp
from jax.experimental import pallas as pl


def kernel(x, edge_index, edge_attr, W_int, b_int, pre_W, pre_b, pre_g, pre_be, pre_a, Wl1, bl1, Wr1, Wl2, bl2, Wr2, post_W, post_b, post_g, post_be, post_a, W_out, b_out):
    raise NotImplementedError("write your pallas kernel here")


# speedup vs baseline: 1.4965x; 1.4965x over previous
import functools

import jax
import jax.numpy as jnp
from jax import lax
from jax.experimental import pallas as pl
from jax.experimental.pallas import tpu as pltpu
from jax.experimental.pallas import tpu_sc as plsc

N = 10000
E = 160000
NC = 2
NS = 16
NW = NC * NS
CH = 128
NCHUNK = 40
EPW = NCHUNK * CH
E_PAD = NW * EPW
RPT = 632
N_PAD = NS * RPT
F32 = jnp.float32


def _mesh():
    return plsc.VectorSubcoreMesh(core_axis_name="c", subcore_axis_name="s",
                                  num_cores=NC, num_subcores=NS)


def _sc_edge_stats(ea_ext, col_r, zeros128):
    @functools.partial(
        pl.kernel,
        mesh=_mesh(),
        out_type=jax.ShapeDtypeStruct((NC, N, 128), F32),
        scratch_types=[
            pltpu.VMEM((NCHUNK, CH), jnp.int32),
            pltpu.VMEM((CH, 128), F32),
            pltpu.MemorySpace.VMEM_SHARED((N_PAD, 128), F32),
        ],
    )
    def k(ea_hbm, col_hbm, z32_hbm, se_hbm, col_v, ebuf, acce):
        c = lax.axis_index("c")
        s = lax.axis_index("s")
        w = s * NC + c
        base = s * RPT
        pltpu.sync_copy(z32_hbm.at[pl.ds(base, RPT)], acce.at[pl.ds(base, RPT)])
        pltpu.sync_copy(col_hbm.at[w], col_v)
        plsc.subcore_barrier()

        def chunk(j, carry):
            pltpu.sync_copy(ea_hbm.at[pl.ds(w * EPW + j * CH, CH)], ebuf)
            pltpu.sync_copy(ebuf, acce.at[col_v.at[j]], add=True)
            return carry

        lax.fori_loop(0, NCHUNK, chunk, 0)
        plsc.subcore_barrier()

        @pl.when(s < NS - 1)
        def _():
            pltpu.sync_copy(acce.at[pl.ds(base, RPT)], se_hbm.at[c, pl.ds(base, RPT)])

        @pl.when(s == NS - 1)
        def _():
            last = N - (NS - 1) * RPT
            pltpu.sync_copy(acce.at[pl.ds(base, last)], se_hbm.at[c, pl.ds(base, last)])

    return k(ea_ext, col_r, zeros128)


def _sc_segsum(tables, row_r, col_r, zeros128):
    P = len(tables)

    @functools.partial(
        pl.kernel,
        mesh=_mesh(),
        out_type=jax.ShapeDtypeStruct((P, NC, N, 128), F32),
        scratch_types=[
            pltpu.VMEM((NCHUNK, CH), jnp.int32),
            pltpu.VMEM((NCHUNK, CH), jnp.int32),
            pltpu.VMEM((CH, 128), F32),
            pltpu.SemaphoreType.DMA,
            pltpu.MemorySpace.VMEM_SHARED((N_PAD, 128), F32),
        ],
    )
    def k(*refs):
        tabs = refs[:P]
        row_hbm, col_hbm, z128_hbm, out_hbm = refs[P:P + 4]
        row_v, col_v, gbuf, sem, acc = refs[P + 4:]
        c = lax.axis_index("c")
        s = lax.axis_index("s")
        w = s * NC + c
        base = s * RPT
        pltpu.sync_copy(row_hbm.at[w], row_v)
        pltpu.sync_copy(col_hbm.at[w], col_v)
        for p in range(P):
            pltpu.sync_copy(z128_hbm.at[pl.ds(base, RPT)], acc.at[pl.ds(base, RPT)])
            plsc.subcore_barrier()

            def chunk(j, carry):
                pltpu.async_copy(tabs[p].at[row_v.at[j]], gbuf, sem).wait()
                pltpu.sync_copy(gbuf, acc.at[col_v.at[j]], add=True)
                return carry

            lax.fori_loop(0, NCHUNK, chunk, 0)
            plsc.subcore_barrier()

            @pl.when(s < NS - 1)
            def _():
                pltpu.sync_copy(acc.at[pl.ds(base, RPT)],
                                out_hbm.at[p, c, pl.ds(base, RPT)])

            @pl.when(s == NS - 1)
            def _():
                last = N - (NS - 1) * RPT
                pltpu.sync_copy(acc.at[pl.ds(base, last)],
                                out_hbm.at[p, c, pl.ds(base, last)])

            plsc.subcore_barrier()

    return k(*tables, row_r, col_r, zeros128)


def _dotT(a, b):
    return lax.dot_general(a.astype(jnp.bfloat16), b.astype(jnp.bfloat16),
                           (((1,), (1,)), ((), ())),
                           preferred_element_type=F32)


def _tc1(x, Wx):
    def body(x_ref, w_ref, o_ref):
        o_ref[...] = _dotT(x_ref[...], w_ref[...])

    return pl.pallas_call(
        body, out_shape=jax.ShapeDtypeStruct((N, 128), F32))(x, Wx)


def _tc1b(ea, We, b_int):
    R = 2000
    G = E // R

    def body(ea_ref, w_ref, b_ref, o_ref):
        o_ref[...] = _dotT(ea_ref[...], w_ref[...]) + b_ref[...]

    return pl.pallas_call(
        body,
        grid=(G,),
        in_specs=[
            pl.BlockSpec((R, 16), lambda i: (i, 0)),
            pl.BlockSpec((128, 16), lambda i: (0, 0)),
            pl.BlockSpec((1, 128), lambda i: (0, 0)),
        ],
        out_specs=pl.BlockSpec((R, 128), lambda i: (i, 0)),
        out_shape=jax.ShapeDtypeStruct((E, 128), F32),
    )(ea, We, b_int)


def _sc_edge_combined(y, ue_ext, row_r, col_r, zeros128):
    @functools.partial(
        pl.kernel,
        mesh=_mesh(),
        out_type=jax.ShapeDtypeStruct((NC, N, 128), F32),
        scratch_types=[
            pltpu.VMEM((NCHUNK, CH), jnp.int32),
            pltpu.VMEM((NCHUNK, CH), jnp.int32),
            pltpu.VMEM((CH, 128), F32),
            pltpu.VMEM((CH, 128), F32),
            pltpu.SemaphoreType.DMA,
            pltpu.MemorySpace.VMEM_SHARED((N_PAD, 128), F32),
        ],
    )
    def k(y_hbm, ue_hbm, row_hbm, col_hbm, z128_hbm, s_hbm,
          row_v, col_v, gbuf, ubuf, sem, acc):
        c = lax.axis_index("c")
        s = lax.axis_index("s")
        w = s * NC + c
        base = s * RPT
        pltpu.sync_copy(z128_hbm.at[pl.ds(base, RPT)], acc.at[pl.ds(base, RPT)])
        pltpu.sync_copy(row_hbm.at[w], row_v)
        pltpu.sync_copy(col_hbm.at[w], col_v)
        plsc.subcore_barrier()

        def chunk(j, carry):
            pltpu.async_copy(y_hbm.at[row_v.at[j]], gbuf, sem).wait()
            pltpu.sync_copy(ue_hbm.at[pl.ds(w * EPW + j * CH, CH)], ubuf)
            pltpu.sync_copy(gbuf, acc.at[col_v.at[j]], add=True)
            pltpu.sync_copy(ubuf, acc.at[col_v.at[j]], add=True)
            return carry

        lax.fori_loop(0, NCHUNK, chunk, 0)
        plsc.subcore_barrier()

        @pl.when(s < NS - 1)
        def _():
            pltpu.sync_copy(acc.at[pl.ds(base, RPT)], s_hbm.at[c, pl.ds(base, RPT)])

        @pl.when(s == NS - 1)
        def _():
            last = N - (NS - 1) * RPT
            pltpu.sync_copy(acc.at[pl.ds(base, last)], s_hbm.at[c, pl.ds(base, last)])

    return k(y, ue_ext, row_r, col_r, zeros128)


def _tc2(sy, se, preW, preb, preg, prebe, prea):
    def body(sy_ref, se_ref, pw_ref, pb_ref, pg_ref,
             pbe_ref, pa_ref, h_ref, inv_ref):
        syt = sy_ref[0] + sy_ref[1]
        cnt_raw = se_ref[0][:, 16:17] + se_ref[1][:, 16:17]
        cnt = jnp.maximum(cnt_raw, 1.0)
        h = syt / cnt
        for i in range(8):
            z = _dotT(h, pw_ref[i]) + pb_ref[i]
            m = jnp.mean(z, axis=0, keepdims=True)
            v = jnp.mean((z - m) ** 2, axis=0, keepdims=True)
            zn = pg_ref[i] * (z - m) / jnp.sqrt(v + 1e-5) + pbe_ref[i]
            h = jnp.where(zn >= 0, zn, pa_ref[i] * zn)
        h_ref[...] = h
        inv_ref[...] = jnp.broadcast_to(cnt, (N, 128))

    return pl.pallas_call(
        body,
        out_shape=[jax.ShapeDtypeStruct((N, 128), F32),
                   jax.ShapeDtypeStruct((N, 128), F32)],
    )(sy, se, preW, preb, preg, prebe, prea)


def _tc3(agg1, h, inv_b, Wl1, Wr1, bl1):
    R = 2000
    G = N // R

    def body(a_ref, h_ref, inv_ref, wl_ref, wr_ref, bl_ref, *o_refs):
        a = (a_ref[0] + a_ref[1]) / inv_ref[...]
        z = _dotT(a, wl_ref[...]) + _dotT(h_ref[...], wr_ref[...]) + bl_ref[...]
        z = jnp.maximum(z, 0.0)
        for c in range(8):
            o_refs[c][...] = z[:, c * 128:(c + 1) * 128]

    return pl.pallas_call(
        body,
        grid=(G,),
        in_specs=[
            pl.BlockSpec((2, R, 128), lambda i: (0, i, 0)),
            pl.BlockSpec((R, 128), lambda i: (i, 0)),
            pl.BlockSpec((R, 128), lambda i: (i, 0)),
            pl.BlockSpec((1024, 128), lambda i: (0, 0)),
            pl.BlockSpec((1024, 128), lambda i: (0, 0)),
            pl.BlockSpec((1, 1024), lambda i: (0, 0)),
        ],
        out_specs=[pl.BlockSpec((R, 128), lambda i: (i, 0))] * 8,
        out_shape=[jax.ShapeDtypeStruct((N, 128), F32)] * 8,
    )(agg1, h, inv_b, Wl1, Wr1, bl1)


def _tc4(agg2, h1p, inv_b, Wl2, Wr2, bl2, pW0, pb0):
    R = 400
    G = N // R

    def body(a_ref, *refs):
        h1_refs = refs[:8]
        inv_ref, wl_ref, wr_ref, bl_ref, pw_ref, pb_ref, z3_ref, st_ref = refs[8:]
        i = pl.program_id(0)
        aggf = jnp.concatenate([a_ref[c, 0] + a_ref[c, 1] for c in range(8)],
                               axis=1)
        h1f = jnp.concatenate([h1_refs[c][...] for c in range(8)], axis=1)
        z = (_dotT(aggf / inv_ref[:, :1], wl_ref[...])
             + _dotT(h1f, wr_ref[...]) + bl_ref[...])
        h2 = jnp.maximum(z, 0.0)
        z3 = _dotT(h2, pw_ref[...]) + pb_ref[...]
        z3_ref[...] = z3

        @pl.when(i == 0)
        def _():
            st_ref[...] = jnp.zeros_like(st_ref)
            st_ref[2:3, :] = jnp.mean(z3, axis=0, keepdims=True)

        d = z3 - st_ref[2:3, :]
        st_ref[0:1, :] += jnp.sum(d, axis=0, keepdims=True)
        st_ref[1:2, :] += jnp.sum(d * d, axis=0, keepdims=True)

    return pl.pallas_call(
        body,
        grid=(G,),
        in_specs=[pl.BlockSpec((8, 2, R, 128), lambda i: (0, 0, i, 0))]
        + [pl.BlockSpec((R, 128), lambda i: (i, 0))] * 8
        + [
            pl.BlockSpec((R, 128), lambda i: (i, 0)),
            pl.BlockSpec((1024, 1024), lambda i: (0, 0)),
            pl.BlockSpec((1024, 1024), lambda i: (0, 0)),
            pl.BlockSpec((1, 1024), lambda i: (0, 0)),
            pl.BlockSpec((1024, 1024), lambda i: (0, 0)),
            pl.BlockSpec((1, 1024), lambda i: (0, 0)),
        ],
        out_specs=[pl.BlockSpec((R, 1024), lambda i: (i, 0)),
                   pl.BlockSpec((8, 1024), lambda i: (0, 0))],
        out_shape=[jax.ShapeDtypeStruct((N, 1024), F32),
                   jax.ShapeDtypeStruct((8, 1024), F32)],
    )(agg2, *h1p, inv_b, Wl2, Wr2, bl2, pW0, pb0)


def _tc5(z3, st3, pg0, pbe0, pa0, pW1, pb1):
    R = 2000
    G = N // R

    def body(z3_ref, st3_ref, pg_ref, pbe_ref, pa_ref, pw_ref, pb_ref,
             z4_ref, st_ref):
        i = pl.program_id(0)
        dm = st3_ref[0:1, :] * (1.0 / N)
        m = st3_ref[2:3, :] + dm
        v = st3_ref[1:2, :] * (1.0 / N) - dm * dm
        zn = pg_ref[...] * (z3_ref[...] - m) / jnp.sqrt(v + 1e-5) + pbe_ref[...]
        h3 = jnp.where(zn >= 0, zn, pa_ref[...] * zn)
        z4 = _dotT(h3, pw_ref[...]) + pb_ref[...]
        z4_ref[...] = z4

        @pl.when(i == 0)
        def _():
            st_ref[...] = jnp.zeros_like(st_ref)
            st_ref[2:3, :] = jnp.mean(z4, axis=0, keepdims=True)

        d = z4 - st_ref[2:3, :]
        st_ref[0:1, :] += jnp.sum(d, axis=0, keepdims=True)
        st_ref[1:2, :] += jnp.sum(d * d, axis=0, keepdims=True)

    return pl.pallas_call(
        body,
        grid=(G,),
        in_specs=[
            pl.BlockSpec((R, 1024), lambda i: (i, 0)),
            pl.BlockSpec((8, 1024), lambda i: (0, 0)),
            pl.BlockSpec((1, 1024), lambda i: (0, 0)),
            pl.BlockSpec((1, 1024), lambda i: (0, 0)),
            pl.BlockSpec((1, 1024), lambda i: (0, 0)),
            pl.BlockSpec((128, 1024), lambda i: (0, 0)),
            pl.BlockSpec((1, 128), lambda i: (0, 0)),
        ],
        out_specs=[pl.BlockSpec((R, 128), lambda i: (i, 0)),
                   pl.BlockSpec((8, 128), lambda i: (0, 0))],
        out_shape=[jax.ShapeDtypeStruct((N, 128), F32),
                   jax.ShapeDtypeStruct((8, 128), F32)],
    )(z3, st3, pg0, pbe0, pa0, pW1, pb1)


def _tc6(z4, st4, pg1, pbe1, pa1, Wo, bo):
    def body(z4_ref, st_ref, pg_ref, pbe_ref, pa_ref, wo_ref, bo_ref, o_ref):
        dm = st_ref[0:1, :] * (1.0 / N)
        m = st_ref[2:3, :] + dm
        v = st_ref[1:2, :] * (1.0 / N) - dm * dm
        zn = pg_ref[...] * (z4_ref[...] - m) / jnp.sqrt(v + 1e-5) + pbe_ref[...]
        h4 = jnp.where(zn >= 0, zn, pa_ref[...] * zn)
        h4_16 = h4.astype(jnp.bfloat16).astype(F32)
        w16 = wo_ref[...].astype(jnp.bfloat16).astype(F32)
        o_ref[...] = jnp.sum(h4_16 * w16, axis=1) + bo_ref[0, 0]

    return pl.pallas_call(
        body, out_shape=jax.ShapeDtypeStruct((N,), F32))(
            z4, st4, pg1, pbe1, pa1, Wo, bo)


def kernel(x, edge_index, edge_attr, W_int, b_int, pre_W, pre_b, pre_g,
           pre_be, pre_a, Wl1, bl1, Wr1, Wl2, bl2, Wr2, post_W, post_b,
           post_g, post_be, post_a, W_out, b_out):
    W_x = W_int[:, :256]
    W_e = W_int[:, 256:]
    row = edge_index[0]
    col = edge_index[1]
    pad = E_PAD - E
    row_r = jnp.concatenate([row, jnp.zeros((pad,), jnp.int32)]).reshape(
        NW, NCHUNK, CH)
    col_r = jnp.concatenate([col, jnp.full((pad,), N, jnp.int32)]).reshape(
        NW, NCHUNK, CH)
    ones_ext = jnp.zeros((E_PAD, 128), F32).at[:, 16].set(1.0)
    z128 = jnp.zeros((N_PAD, 128), F32)

    preW = jnp.stack(pre_W)
    preb = jnp.stack(pre_b)
    preg = jnp.stack(pre_g)
    prebe = jnp.stack(pre_be)
    prea = jnp.broadcast_to(jnp.stack(pre_a)[:, None], (8, 128))

    y = _tc1(x, W_x)
    ue = _tc1b(edge_attr, W_e, b_int.reshape(1, 128))
    ue_ext = jnp.concatenate([ue, jnp.zeros((E_PAD - E, 128), F32)])
    sy = _sc_edge_combined(y, ue_ext, row_r, col_r, z128)
    se = _sc_edge_stats(ones_ext, col_r, z128)
    h, inv_b = _tc2(sy, se, preW, preb, preg, prebe, prea)
    agg1 = _sc_segsum([h], row_r, col_r, z128)[0]
    h1p = _tc3(agg1, h, inv_b, Wl1, Wr1, bl1.reshape(1, 1024))
    agg2 = _sc_segsum(h1p, row_r, col_r, z128)
    z3, st3 = _tc4(agg2, h1p, inv_b, Wl2, Wr2, bl2.reshape(1, 1024), post_W[0],
                   post_b[0].reshape(1, 1024))
    z4, st4 = _tc5(z3, st3, post_g[0].reshape(1, 1024),
                   post_be[0].reshape(1, 1024),
                   jnp.broadcast_to(post_a[0], (1, 1024)), post_W[1],
                   post_b[1].reshape(1, 128))
    out = _tc6(z4, st4, post_g[1].reshape(1, 128), post_be[1].reshape(1, 128),
               jnp.broadcast_to(post_a[1], (1, 128)), W_out,
               b_out.reshape(1, 1))
    return out
